# baseline (speedup 1.0000x reference)
import jax
import jax.numpy as jnp
from jax.experimental import pallas as pl
from jax.experimental.pallas import tpu as pltpu


def _row_dot(s_ref, m_ref, t_ref, o_ref):
    o_ref[0, 0] = jnp.sum(m_ref[0] * t_ref[...])


def kernel(M, theta, state):
    s = jnp.asarray(state, jnp.int32).reshape(1)
    m3 = M.reshape(M.shape[0], 1, M.shape[1])
    t2 = theta.reshape(1, theta.shape[0])
    grid_spec = pltpu.PrefetchScalarGridSpec(
        num_scalar_prefetch=1,
        grid=(1,),
        in_specs=[
            pl.BlockSpec((1, 1, M.shape[1]), lambda i, s: (s[0], 0, 0)),
            pl.BlockSpec((1, M.shape[1]), lambda i, s: (0, 0)),
        ],
        out_specs=pl.BlockSpec(memory_space=pltpu.SMEM),
    )
    out = pl.pallas_call(
        _row_dot,
        grid_spec=grid_spec,
        out_shape=jax.ShapeDtypeStruct((1, 1), jnp.float32),
    )(s, m3, t2)
    return out.reshape(())


# speedup vs baseline: 1.2961x; 1.2961x over previous
import jax
import jax.numpy as jnp
from jax.experimental import pallas as pl
from jax.experimental.pallas import tpu as pltpu


def _row_dot(s_ref, m_ref, t_ref, o_ref):
    i = s_ref[0]
    acc = m_ref[i, 0] * t_ref[0]
    for j in range(1, 7):
        acc += m_ref[i, j] * t_ref[j]
    o_ref[0] = acc


def kernel(M, theta, state):
    s = jnp.asarray(state, jnp.int32).reshape(1)
    out = pl.pallas_call(
        _row_dot,
        out_shape=jax.ShapeDtypeStruct((1,), jnp.float32),
        in_specs=[
            pl.BlockSpec(memory_space=pltpu.SMEM),
            pl.BlockSpec(memory_space=pltpu.SMEM),
            pl.BlockSpec(memory_space=pltpu.SMEM),
        ],
        out_specs=pl.BlockSpec(memory_space=pltpu.SMEM),
    )(s, M, theta)
    return out.reshape(())
